# baseline (speedup 1.0000x reference)
import jax
import jax.numpy as jnp
from jax.experimental import pallas as pl
from jax.experimental.pallas import tpu as pltpu

N = 5000
D_IN = 12544
D_HID = 1024
BM = 200
NM = N // BM


def _body(x_ref, w1_ref, b1_ref, w2_ref, b2_ref, wh_ref, bh_ref, out_ref):
    x = x_ref[...].astype(jnp.bfloat16)
    h1 = jnp.maximum(
        jnp.dot(x, w1_ref[...], preferred_element_type=jnp.float32)
        + b1_ref[...], 0.0)
    h2 = jnp.maximum(
        jnp.dot(h1.astype(jnp.bfloat16), w2_ref[...],
                preferred_element_type=jnp.float32)
        + b2_ref[...], 0.0)
    out_ref[...] = (
        jnp.dot(h2.astype(jnp.bfloat16), wh_ref[...],
                preferred_element_type=jnp.float32)
        + bh_ref[...])


def kernel(feature_vectors, W1, b1, W2, b2, Wc, bc, Wr, br):
    wh = jnp.concatenate([Wc, Wr], axis=1).astype(jnp.bfloat16)
    bh = jnp.concatenate([bc, br])[None, :]
    w1 = W1.astype(jnp.bfloat16)
    w2 = W2.astype(jnp.bfloat16)
    b1r = b1[None, :]
    b2r = b2[None, :]
    n_heads = wh.shape[1]

    out = pl.pallas_call(
        _body,
        grid=(NM,),
        in_specs=[
            pl.BlockSpec((BM, D_IN), lambda m: (m, 0)),
            pl.BlockSpec((D_IN, D_HID), lambda m: (0, 0)),
            pl.BlockSpec((1, D_HID), lambda m: (0, 0)),
            pl.BlockSpec((D_HID, D_HID), lambda m: (0, 0)),
            pl.BlockSpec((1, D_HID), lambda m: (0, 0)),
            pl.BlockSpec((D_HID, n_heads), lambda m: (0, 0)),
            pl.BlockSpec((1, n_heads), lambda m: (0, 0)),
        ],
        out_specs=pl.BlockSpec((BM, n_heads), lambda m: (m, 0)),
        out_shape=jax.ShapeDtypeStruct((N, n_heads), jnp.float32),
        compiler_params=pltpu.CompilerParams(
            vmem_limit_bytes=100 * 1024 * 1024),
    )(feature_vectors, w1, b1r, w2, b2r, wh, bh)

    return out[:, :4], out[:, 4:]
